# Initial kernel scaffold; baseline (speedup 1.0000x reference)
import jax
import jax.numpy as jnp
from jax.experimental import pallas as pl


def kernel(token_ids, weight):
    raise NotImplementedError("write your pallas kernel here")


# speedup vs baseline: 1.8461x; 1.8461x over previous
import functools

import jax
import jax.numpy as jnp
from jax import lax
from jax.experimental import pallas as pl
from jax.experimental.pallas import tpu as pltpu
from jax.experimental.pallas import tpu_sc as plsc

_D = 64
_IDX_ROW = 128


@functools.lru_cache(maxsize=None)
def _build(B):
    info = plsc.get_sparse_core_info()
    nw = info.num_cores * info.num_subcores
    b_per_w = B // nw
    chunk = 1024
    n_chunks = b_per_w // chunk
    sub = chunk // _IDX_ROW
    mesh = plsc.VectorSubcoreMesh(core_axis_name="c", subcore_axis_name="s")

    @functools.partial(
        pl.kernel,
        mesh=mesh,
        out_type=jax.ShapeDtypeStruct((B, _D), jnp.float32),
        compiler_params=pltpu.CompilerParams(use_tc_tiling_on_sc=False),
        scratch_types=[
            pltpu.VMEM((sub, _IDX_ROW), jnp.int32),
            pltpu.VMEM((chunk, _D), jnp.float32),
            pltpu.SemaphoreType.DMA,
        ],
    )
    def gather_kernel(idx_hbm, table_hbm, out_hbm, idx_v, rows_v, sem):
        wid = lax.axis_index("s") * info.num_cores + lax.axis_index("c")
        row0 = wid * (b_per_w // _IDX_ROW)
        base = wid * b_per_w

        def body(g, carry):
            pltpu.sync_copy(idx_hbm.at[pl.ds(row0 + g * sub, sub)], idx_v)
            copies = [
                pltpu.async_copy(
                    table_hbm.at[idx_v.at[j]],
                    rows_v.at[pl.ds(j * _IDX_ROW, _IDX_ROW)],
                    sem,
                )
                for j in range(sub)
            ]
            for c in copies:
                c.wait()
            pltpu.sync_copy(rows_v, out_hbm.at[pl.ds(base + g * chunk, chunk)])
            return carry

        lax.fori_loop(0, n_chunks, body, 0)

    return gather_kernel


def kernel(token_ids, weight):
    s, t = token_ids.shape
    b = s * t
    idx = token_ids.reshape(b // _IDX_ROW, _IDX_ROW).astype(jnp.int32)
    out = _build(b)(idx, weight)
    return out.reshape(s, t, weight.shape[1])


# speedup vs baseline: 1.8695x; 1.0127x over previous
import functools

import jax
import jax.numpy as jnp
from jax import lax
from jax.experimental import pallas as pl
from jax.experimental.pallas import tpu as pltpu
from jax.experimental.pallas import tpu_sc as plsc

_D = 64
_IDX_ROW = 128


@functools.lru_cache(maxsize=None)
def _build(B):
    info = plsc.get_sparse_core_info()
    nw = info.num_cores * info.num_subcores
    b_per_w = B // nw
    rows_per_w = b_per_w // _IDX_ROW
    chunk = 512
    n_chunks = b_per_w // chunk
    sub = chunk // _IDX_ROW
    mesh = plsc.VectorSubcoreMesh(core_axis_name="c", subcore_axis_name="s")

    @functools.partial(
        pl.kernel,
        mesh=mesh,
        out_type=jax.ShapeDtypeStruct((B, _D), jnp.float32),
        compiler_params=pltpu.CompilerParams(use_tc_tiling_on_sc=False),
        scratch_types=[
            pltpu.VMEM((rows_per_w, _IDX_ROW), jnp.int32),
            pltpu.VMEM((chunk, _D), jnp.float32),
            pltpu.VMEM((chunk, _D), jnp.float32),
            pltpu.SemaphoreType.DMA,
            pltpu.SemaphoreType.DMA,
            pltpu.SemaphoreType.DMA,
        ],
    )
    def gather_kernel(idx_hbm, table_hbm, out_hbm, idx_v, rows0, rows1,
                      gsem, wsem0, wsem1):
        wid = lax.axis_index("s") * info.num_cores + lax.axis_index("c")
        base = wid * b_per_w
        pltpu.sync_copy(idx_hbm.at[pl.ds(wid * rows_per_w, rows_per_w)], idx_v)
        rows = (rows0, rows1)
        wsems = (wsem0, wsem1)

        def pair_body(i, carry):
            for b in range(2):
                g = i * 2 + b

                @pl.when(i > 0)
                def _(b=b):
                    pltpu.make_async_copy(
                        rows[b], out_hbm.at[pl.ds(0, chunk)], wsems[b]
                    ).wait()

                copies = [
                    pltpu.async_copy(
                        table_hbm.at[idx_v.at[g * sub + j]],
                        rows[b].at[pl.ds(j * _IDX_ROW, _IDX_ROW)],
                        gsem,
                    )
                    for j in range(sub)
                ]
                for c in copies:
                    c.wait()
                pltpu.async_copy(
                    rows[b], out_hbm.at[pl.ds(base + g * chunk, chunk)], wsems[b]
                )
            return carry

        lax.fori_loop(0, n_chunks // 2, pair_body, 0)
        pltpu.make_async_copy(rows0, out_hbm.at[pl.ds(0, chunk)], wsem0).wait()
        pltpu.make_async_copy(rows1, out_hbm.at[pl.ds(0, chunk)], wsem1).wait()

    return gather_kernel


def kernel(token_ids, weight):
    s, t = token_ids.shape
    b = s * t
    idx = token_ids.reshape(b // _IDX_ROW, _IDX_ROW).astype(jnp.int32)
    out = _build(b)(idx, weight)
    return out.reshape(s, t, weight.shape[1])


# speedup vs baseline: 2.9201x; 1.5620x over previous
import functools

import jax
import jax.numpy as jnp
from jax import lax
from jax.experimental import pallas as pl
from jax.experimental.pallas import tpu as pltpu
from jax.experimental.pallas import tpu_sc as plsc

_D = 64
_BLK = 128


@functools.lru_cache(maxsize=None)
def _build(n_tok, n_seq):
    info = plsc.get_sparse_core_info()
    nw = info.num_cores * info.num_subcores
    n_blocks = n_tok * n_seq // _BLK
    blocks_per_w = n_blocks // nw
    mesh = plsc.VectorSubcoreMesh(core_axis_name="c", subcore_axis_name="s")

    @functools.partial(
        pl.kernel,
        mesh=mesh,
        out_type=jax.ShapeDtypeStruct((n_seq, _D // 8, n_tok // _BLK, 8 * _BLK),
                                      jnp.float32),
        compiler_params=pltpu.CompilerParams(use_tc_tiling_on_sc=False,
                                             needs_layout_passes=False),
        scratch_types=[
            pltpu.VMEM((blocks_per_w, _BLK), jnp.int32),
            pltpu.VMEM((_BLK, _D), jnp.float32),
            pltpu.VMEM((_BLK, _D), jnp.float32),
            pltpu.VMEM((_BLK, _D + 1), jnp.float32),
            pltpu.VMEM((_D // 8, 8 * _BLK), jnp.float32),
            pltpu.VMEM((_D // 8, 8 * _BLK), jnp.float32),
            pltpu.SemaphoreType.DMA,
            pltpu.SemaphoreType.DMA,
            pltpu.SemaphoreType.DMA,
            pltpu.SemaphoreType.DMA,
        ],
    )
    def gather_kernel(tids_hbm, table_hbm, out_hbm,
                      idx_v, buf0, buf1, bufsk, tp0, tp1,
                      gsem0, gsem1, wsem0, wsem1):
        wid = lax.axis_index("s") * info.num_cores + lax.axis_index("c")
        blk0 = wid * blocks_per_w
        bufs = (buf0, buf1)
        tps = (tp0, tp1)
        gsems = (gsem0, gsem1)
        wsems = (wsem0, wsem1)
        iota = lax.iota(jnp.int32, 16)
        rows = [iota + (16 * c) for c in range(_BLK // 16)]

        pltpu.sync_copy(tids_hbm.at[pl.ds(blk0, blocks_per_w)], idx_v)

        def fire_gather(k, buf, sem):
            pltpu.async_copy(table_hbm.at[idx_v.at[k]], buf, sem)

        def transpose(buf, tp):
            @plsc.parallel_loop(0, _BLK // 4, unroll=4)
            def skew_body(s):
                for u in range(4):
                    tok = s * 4 + u
                    for q in range(_D // 16):
                        bufsk[tok, pl.ds(16 * q, 16)] = buf[tok, pl.ds(16 * q, 16)]

            @plsc.parallel_loop(0, _D, unroll=4)
            def col_body(d):
                cols = jnp.full((16,), d, dtype=jnp.int32)
                dg = d // 8
                off = (d % 8) * _BLK
                for c in range(_BLK // 16):
                    v = plsc.load_gather(bufsk, [rows[c], cols])
                    tp[dg, pl.ds(off + 16 * c, 16)] = v

        def write_out(k, tp, sem):
            blk = blk0 + k
            t = blk // (n_tok // _BLK)
            bg = blk - t * (n_tok // _BLK)
            pltpu.async_copy(tp, out_hbm.at[t, :, bg], sem)

        fire_gather(0, buf0, gsem0)

        def pair_body(i, carry):
            for b in range(2):
                k = i * 2 + b

                if b == 0:
                    fire_gather(k + 1, bufs[1 - b], gsems[1 - b])
                else:
                    @pl.when(i < blocks_per_w // 2 - 1)
                    def _(b=b, k=k):
                        fire_gather(k + 1, bufs[1 - b], gsems[1 - b])

                pltpu.make_async_copy(
                    table_hbm.at[idx_v.at[k]], bufs[b], gsems[b]
                ).wait()

                @pl.when(i >= 1)
                def _(b=b):
                    pltpu.make_async_copy(
                        tps[b], out_hbm.at[0, :, 0], wsems[b]
                    ).wait()

                transpose(bufs[b], tps[b])
                write_out(k, tps[b], wsems[b])
            return carry

        lax.fori_loop(0, blocks_per_w // 2, pair_body, 0)
        pltpu.make_async_copy(tp0, out_hbm.at[0, :, 0], wsem0).wait()
        pltpu.make_async_copy(tp1, out_hbm.at[0, :, 0], wsem1).wait()

    return gather_kernel


def kernel(token_ids, weight):
    n_tok, n_seq = token_ids.shape
    tids = token_ids.T.reshape(n_tok * n_seq // _BLK, _BLK).astype(jnp.int32)
    out4 = _build(n_tok, n_seq)(tids, weight)
    out5 = out4.reshape(n_seq, _D // 8, n_tok // _BLK, 8, _BLK)
    return out5.transpose(2, 4, 0, 1, 3).reshape(n_tok, n_seq, _D)
